# baseline (speedup 1.0000x reference)
import functools

import jax
import jax.numpy as jnp
from jax import lax
from jax.experimental import pallas as pl
from jax.experimental.pallas import tpu as pltpu
from jax.experimental.pallas import tpu_sc as plsc

N = 10000
E = 320000
D = 128
NG = 64

NC = 2
NS = 16
NW = NC * NS

LANES = 128
STEPS = 80
EPT = STEPS * LANES
E_PAD = NW * EPT
ACC_ROWS = 10112
RPT = ACC_ROWS // NS
TRASH = N
N_TRASH = ACC_ROWS - N

_MESH = plsc.VectorSubcoreMesh(core_axis_name="c", subcore_axis_name="s")


_NBUF = 3
_SCATTER_OUT = jax.ShapeDtypeStruct((NC, ACC_ROWS, D), jnp.float32)
_SCATTER_SCRATCH = [
    [pltpu.VMEM((LANES,), jnp.int32) for _ in range(_NBUF)],
    [pltpu.VMEM((LANES,), jnp.int32) for _ in range(_NBUF)],
    [pltpu.VMEM((LANES, D), jnp.float32) for _ in range(_NBUF)],
    pltpu.MemorySpace.VMEM_SHARED((ACC_ROWS, D), jnp.float32),
    [pltpu.SemaphoreType.DMA for _ in range(_NBUF)],
    [pltpu.SemaphoreType.DMA for _ in range(_NBUF)],
]


def _sc_scatter_rows_body(g_hbm, src_hbm, dst_hbm, z_hbm, out_hbm,
                          src_b, dst_b, rows, acc, isems, gsems):
    c = lax.axis_index("c")
    s = lax.axis_index("s")
    wid = s * NC + c
    base = wid * EPT

    def fire_idx(j, b):
        pltpu.async_copy(src_hbm.at[pl.ds(base + j * LANES, LANES)],
                         src_b[b], isems[b])
        pltpu.async_copy(dst_hbm.at[pl.ds(base + j * LANES, LANES)],
                         dst_b[b], isems[b])

    def wait_idx(b):
        pltpu.make_async_copy(src_hbm.at[pl.ds(base, LANES)],
                              src_b[b], isems[b]).wait()
        pltpu.make_async_copy(dst_hbm.at[pl.ds(base, LANES)],
                              dst_b[b], isems[b]).wait()

    def fire_gather(b):
        pltpu.async_copy(g_hbm.at[src_b[b]], rows[b], gsems[b])

    def wait_gather(b):
        pltpu.make_async_copy(g_hbm.at[src_b[b]], rows[b], gsems[b]).wait()

    for b in range(_NBUF):
        fire_idx(b, b)
    pltpu.sync_copy(z_hbm, acc.at[pl.ds(s * RPT, RPT)])
    wait_idx(0)
    fire_gather(0)
    wait_idx(1)
    fire_gather(1)
    plsc.subcore_barrier()

    def step(j, b):
        wait_idx((b + 2) % _NBUF)
        fire_gather((b + 2) % _NBUF)
        wait_gather(b)
        pltpu.sync_copy(rows[b], acc.at[dst_b[b]], add=True)
        fire_idx(j + _NBUF, b)

    def body(k, carry):
        j0 = k * _NBUF
        for b in range(_NBUF):
            step(j0 + b, b)
        return carry

    n_main = (STEPS - 4) // _NBUF
    lax.fori_loop(0, n_main, body, 0, unroll=False)
    for j in range(n_main * _NBUF, STEPS):
        b = j % _NBUF
        if j + 2 < STEPS:
            wait_idx((b + 2) % _NBUF)
            fire_gather((b + 2) % _NBUF)
        wait_gather(b)
        pltpu.sync_copy(rows[b], acc.at[dst_b[b]], add=True)
        if j + _NBUF < STEPS:
            fire_idx(j + _NBUF, b)
    plsc.subcore_barrier()
    pltpu.sync_copy(acc.at[pl.ds(s * RPT, RPT)],
                    out_hbm.at[c].at[pl.ds(s * RPT, RPT)])


_sc_scatter_rows = functools.partial(
    pl.kernel, out_type=_SCATTER_OUT, mesh=_MESH,
    scratch_types=_SCATTER_SCRATCH)(_sc_scatter_rows_body)


HIST = NW * STEPS * 4
_DEG_OUT = jax.ShapeDtypeStruct((NC, NS, HIST), jnp.float32)
NSUB = 8
_DEG_SCRATCH = [
    pltpu.VMEM((EPT,), jnp.int32),
    pltpu.VMEM((NSUB * HIST,), jnp.float32),
]


def _sc_degree_body(dst_hbm, z_hbm, out_hbm, dst_v, hist):
    c = lax.axis_index("c")
    s = lax.axis_index("s")
    wid = s * NC + c
    pltpu.sync_copy(dst_hbm.at[pl.ds(wid * EPT, EPT)], dst_v)
    for m in range(NSUB):
        pltpu.sync_copy(z_hbm, hist.at[pl.ds(m * HIST, HIST)])
    ones16 = jnp.ones((16,), jnp.float32)
    iota16 = lax.iota(jnp.int32, 16)
    lane_off = (iota16 % NSUB) * HIST
    lo_mask = iota16 < 8
    hi_mask = iota16 >= 8

    def body(i, carry):
        idx = dst_v[pl.ds(i * 16, 16)] + lane_off
        plsc.addupdate_scatter(hist, [idx], ones16, mask=lo_mask)
        plsc.addupdate_scatter(hist, [idx], ones16, mask=hi_mask)
        return carry

    lax.fori_loop(0, EPT // 16, body, 0, unroll=False)

    def rbody(i, carry):
        v = hist[pl.ds(i * 16, 16)]
        for m in range(1, NSUB):
            v = v + hist[pl.ds(m * HIST + i * 16, 16)]
        hist[pl.ds(i * 16, 16)] = v
        return carry

    lax.fori_loop(0, HIST // 16, rbody, 0, unroll=False)
    pltpu.sync_copy(hist.at[pl.ds(0, HIST)], out_hbm.at[c].at[s])


_sc_degree = functools.partial(
    pl.kernel, out_type=_DEG_OUT, mesh=_MESH,
    compiler_params=pltpu.CompilerParams(needs_layout_passes=False),
    scratch_types=_DEG_SCRATCH)(_sc_degree_body)


def _tc_dinv_body(degp_ref, dinv_ref):
    deg = jnp.sum(degp_ref[...], axis=(0, 1)) + 1.0
    row = lax.rsqrt(deg).reshape(1, HIST)
    col16 = lax.dot_general(row, jnp.ones((1, 16), jnp.float32),
                            (((0,), (0,)), ((), ())),
                            preferred_element_type=jnp.float32)
    dinv_ref[...] = col16[:N]


def _tc_dinv(degp):
    return pl.pallas_call(
        _tc_dinv_body,
        grid=(1,),
        in_specs=[pl.BlockSpec((NC, NS, HIST), lambda i: (0, 0, 0))],
        out_specs=pl.BlockSpec((N, 16), lambda i: (0, 0)),
        out_shape=jax.ShapeDtypeStruct((N, 16), jnp.float32),
    )(degp)


_BLK = 1000


def _deg_spec():
    return pl.BlockSpec((_BLK, 16), lambda i: (i, 0))


def _s_spec():
    return pl.BlockSpec((NC, _BLK, D), lambda i: (0, i, 0))


def _dinv_block(dinv_ref):
    return dinv_ref[:, 0:1]


def _tc_g1_body(degp_ref, x_ref, w_ref, g_ref):
    dinv = _dinv_block(degp_ref)
    g_ref[...] = dinv * jnp.dot(x_ref[...], w_ref[...],
                                preferred_element_type=jnp.float32)


def _tc_g1(degp, x, W1):
    return pl.pallas_call(
        _tc_g1_body,
        grid=(N // _BLK,),
        in_specs=[
            _deg_spec(),
            pl.BlockSpec((_BLK, D), lambda i: (i, 0)),
            pl.BlockSpec((D, D), lambda i: (0, 0)),
        ],
        out_specs=pl.BlockSpec((_BLK, D), lambda i: (i, 0)),
        out_shape=jax.ShapeDtypeStruct((N, D), jnp.float32),
    )(degp, x, W1)


def _tc_g2_body(degp_ref, s_ref, g1_ref, b1_ref, w2_ref, g2_ref):
    dinv = _dinv_block(degp_ref)
    h1 = dinv * (s_ref[0] + s_ref[1] + g1_ref[...]) + b1_ref[...]
    h1 = jnp.maximum(h1, 0.0)
    g2_ref[...] = dinv * jnp.dot(h1, w2_ref[...],
                                 preferred_element_type=jnp.float32)


def _tc_g2(degp, S1, g1, b1r, W2):
    return pl.pallas_call(
        _tc_g2_body,
        grid=(N // _BLK,),
        in_specs=[
            _deg_spec(),
            _s_spec(),
            pl.BlockSpec((_BLK, D), lambda i: (i, 0)),
            pl.BlockSpec((1, D), lambda i: (0, 0)),
            pl.BlockSpec((D, D), lambda i: (0, 0)),
        ],
        out_specs=pl.BlockSpec((_BLK, D), lambda i: (i, 0)),
        out_shape=jax.ShapeDtypeStruct((N, D), jnp.float32),
    )(degp, S1, g1, b1r, W2)


def _tc_head_body(degp_ref, s_ref, g2_ref, b2_ref,
                  batch_ref, wm1_ref, bm1_ref, wm2_ref, bm2_ref, out_ref):
    dinv = degp_ref[:, 0:1]
    h2 = dinv * (s_ref[0] + s_ref[1] + g2_ref[...]) + b2_ref[...]
    gid = lax.broadcasted_iota(jnp.int32, (NG, N), 0).astype(jnp.float32)
    onehot = (batch_ref[...] == gid).astype(jnp.float32)
    sums = jnp.dot(onehot, h2, preferred_element_type=jnp.float32)
    counts = jnp.sum(onehot, axis=1, keepdims=True)
    pooled = sums / jnp.maximum(counts, 1.0)
    z = jnp.dot(pooled, wm1_ref[...], preferred_element_type=jnp.float32)
    z = jnp.maximum(z + bm1_ref[...], 0.0)
    out_ref[...] = (jnp.sum(z * wm2_ref[...], axis=1, keepdims=True)
                    + bm2_ref[...])


def _tc_head(degp, S2, g2, b2r, batchf, Wm1, bm1r, wm2r, bm2r):
    return pl.pallas_call(
        _tc_head_body,
        grid=(1,),
        in_specs=[
            pl.BlockSpec((N, 16), lambda i: (0, 0)),
            pl.BlockSpec((NC, N, D), lambda i: (0, 0, 0)),
            pl.BlockSpec((N, D), lambda i: (0, 0)),
            pl.BlockSpec((1, D), lambda i: (0, 0)),
            pl.BlockSpec((1, N), lambda i: (0, 0)),
            pl.BlockSpec((D, 16), lambda i: (0, 0)),
            pl.BlockSpec((1, 16), lambda i: (0, 0)),
            pl.BlockSpec((1, 16), lambda i: (0, 0)),
            pl.BlockSpec((1, 1), lambda i: (0, 0)),
        ],
        out_specs=pl.BlockSpec((NG, 1), lambda i: (0, 0)),
        out_shape=jax.ShapeDtypeStruct((NG, 1), jnp.float32),
    )(degp, S2, g2, b2r, batchf, Wm1, bm1r, wm2r, bm2r)


def kernel(x, edge_index, batch, W1, b1, W2, b2, Wm1, bm1, Wm2, bm2):
    src = edge_index[0]
    dst = edge_index[1]
    pad = E_PAD - E
    it = jnp.arange(pad, dtype=jnp.int32)
    src_p = jnp.concatenate([src, it % N])
    dst_p = jnp.concatenate([dst, TRASH + (it % N_TRASH)])
    zeros_d = jnp.zeros((RPT, D), jnp.float32)
    zeros_h = jnp.zeros((HIST,), jnp.float32)

    degp = _sc_degree(dst_p, zeros_h)
    dinv = _tc_dinv(degp)

    g1 = _tc_g1(dinv, x, W1)
    S1 = _sc_scatter_rows(g1, src_p, dst_p, zeros_d)
    g2 = _tc_g2(dinv, S1, g1, b1.reshape(1, D), W2)
    S2 = _sc_scatter_rows(g2, src_p, dst_p, zeros_d)
    out = _tc_head(dinv, S2, g2,
                   b2.reshape(1, D),
                   batch.astype(jnp.float32).reshape(1, N),
                   Wm1, bm1.reshape(1, 16),
                   Wm2.reshape(1, 16), bm2.reshape(1, 1))
    return out.reshape(-1)
